# baseline (speedup 1.0000x reference)
import functools

import jax
import jax.numpy as jnp
from jax import lax
from jax.experimental import pallas as pl
from jax.experimental.pallas import tpu as pltpu
from jax.experimental.pallas import tpu_sc as plsc

VOCAB_DIM = 100000
SEQ_LEN = 200
D_MODEL = 128
BATCH = 1024

NC = 2
NS = 16
NW = NC * NS
CHUNK = SEQ_LEN // 2
NCHUNK = BATCH * 2 // NW
NBUF = 4
LOOKAHEAD = 2


def _sc_body(data_hbm, tok_hbm, pos_hbm, out_hbm,
             idx_all, rows0, rows1, rows2, rows3, pos_v,
             g0, g1, g2, g3, o0, o1, o2, o3):
    wid = lax.axis_index("s") * NC + lax.axis_index("c")
    base = wid * NCHUNK
    rows_v = (rows0, rows1, rows2, rows3)
    gsem = (g0, g1, g2, g3)
    osem = (o0, o1, o2, o3)

    pltpu.sync_copy(data_hbm.at[pl.ds(base, NCHUNK)], idx_all)
    pltpu.sync_copy(pos_hbm, pos_v)

    def issue_gather(c, b):
        pltpu.async_copy(tok_hbm.at[idx_all.at[c]], rows_v[b], gsem[b])

    def wait_gather(c, b):
        pltpu.make_async_copy(tok_hbm.at[idx_all.at[c]], rows_v[b],
                              gsem[b]).wait()

    def wait_out(b):
        pltpu.make_async_copy(rows_v[b], out_hbm.at[base], osem[b]).wait()

    for c0 in range(LOOKAHEAD):
        issue_gather(c0, c0)

    def group(g, carry):
        for b in range(NBUF):
            c = g * NBUF + b
            wait_gather(c, b)

            bn = (b + LOOKAHEAD) % NBUF

            @pl.when(c >= NBUF - LOOKAHEAD)
            def _():
                wait_out(bn)

            @pl.when(c + LOOKAHEAD < NCHUNK)
            def _():
                issue_gather(c + LOOKAHEAD, bn)

            s_base = (b % 2) * CHUNK

            @plsc.parallel_loop(0, CHUNK, step=1, unroll=5)
            def addrow(i):
                for j in range(D_MODEL // 16):
                    sl = pl.ds(j * 16, 16)
                    rows_v[b][i, sl] = rows_v[b][i, sl] + pos_v[s_base + i, sl]

            pltpu.async_copy(rows_v[b], out_hbm.at[c + base], osem[b])
        return carry

    lax.fori_loop(0, NCHUNK // NBUF, group, 0)
    for k in range(LOOKAHEAD):
        wait_out((NCHUNK - LOOKAHEAD + k) % NBUF)


def kernel(data, token_table, pos_table):
    data2 = data.reshape(BATCH * 2, CHUNK).astype(jnp.int32)
    mesh = plsc.VectorSubcoreMesh(core_axis_name="c", subcore_axis_name="s")
    run = functools.partial(
        pl.kernel,
        out_type=jax.ShapeDtypeStruct((BATCH * 2, CHUNK, D_MODEL),
                                      jnp.float32),
        mesh=mesh,
        scratch_types=[
            pltpu.VMEM((NCHUNK, CHUNK), jnp.int32),
            pltpu.VMEM((CHUNK, D_MODEL), jnp.float32),
            pltpu.VMEM((CHUNK, D_MODEL), jnp.float32),
            pltpu.VMEM((CHUNK, D_MODEL), jnp.float32),
            pltpu.VMEM((CHUNK, D_MODEL), jnp.float32),
            pltpu.VMEM((SEQ_LEN, D_MODEL), jnp.float32),
            pltpu.SemaphoreType.DMA,
            pltpu.SemaphoreType.DMA,
            pltpu.SemaphoreType.DMA,
            pltpu.SemaphoreType.DMA,
            pltpu.SemaphoreType.DMA,
            pltpu.SemaphoreType.DMA,
            pltpu.SemaphoreType.DMA,
            pltpu.SemaphoreType.DMA,
        ],
    )(_sc_body)
    out = run(data2, token_table, pos_table)
    return out.reshape(BATCH, SEQ_LEN, D_MODEL)


# speedup vs baseline: 1.8967x; 1.8967x over previous
import functools

import jax
import jax.numpy as jnp
from jax import lax
from jax.experimental import pallas as pl
from jax.experimental.pallas import tpu as pltpu
from jax.experimental.pallas import tpu_sc as plsc

VOCAB_DIM = 100000
SEQ_LEN = 200
D_MODEL = 128
BATCH = 1024

NC = 2
NS = 16
NW = NC * NS
NCHUNK = BATCH // NW
HALF = SEQ_LEN // 2
NBUF = 3
NGROUP = NCHUNK // NBUF
NREM = NCHUNK - NGROUP * NBUF


def _sc_body(data_hbm, tok_hbm, pos_hbm, out_hbm,
             idx_all, rows0, rows1, rows2, pos_v, g0, g1, g2, o0, o1, o2):
    wid = lax.axis_index("s") * NC + lax.axis_index("c")
    base = wid * NCHUNK
    rows_v = (rows0, rows1, rows2)
    gsem = (g0, g1, g2)
    osem = (o0, o1, o2)

    pltpu.sync_copy(data_hbm.at[pl.ds(base, NCHUNK)], idx_all)
    pltpu.sync_copy(pos_hbm, pos_v)

    def issue_gather(c, b):
        pltpu.async_copy(tok_hbm.at[idx_all.at[c, 0]],
                         rows_v[b].at[pl.ds(0, HALF)], gsem[b])
        pltpu.async_copy(tok_hbm.at[idx_all.at[c, 1]],
                         rows_v[b].at[pl.ds(HALF, HALF)], gsem[b])

    def wait_gather(c, b):
        pltpu.make_async_copy(tok_hbm.at[idx_all.at[c, 0]],
                              rows_v[b].at[pl.ds(0, HALF)], gsem[b]).wait()
        pltpu.make_async_copy(tok_hbm.at[idx_all.at[c, 1]],
                              rows_v[b].at[pl.ds(HALF, HALF)], gsem[b]).wait()

    def wait_out(b):
        pltpu.make_async_copy(rows_v[b], out_hbm.at[base], osem[b]).wait()

    def step(c, b):
        wait_gather(c, b)
        bn = (b + 1) % NBUF

        if isinstance(c, int):
            if c >= NBUF - 1:
                wait_out(bn)
            if c + 1 < NCHUNK:
                issue_gather(c + 1, bn)
        else:
            @pl.when(c >= NBUF - 1)
            def _():
                wait_out(bn)

            @pl.when(c + 1 < NCHUNK)
            def _():
                issue_gather(c + 1, bn)

        @plsc.parallel_loop(0, SEQ_LEN, step=1, unroll=5)
        def addrow(i):
            for j in range(D_MODEL // 16):
                sl = pl.ds(j * 16, 16)
                rows_v[b][i, sl] = rows_v[b][i, sl] + pos_v[i, sl]

        pltpu.async_copy(rows_v[b], out_hbm.at[base + c], osem[b])

    issue_gather(0, 0)

    def group(g, carry):
        for b in range(NBUF):
            step(g * NBUF + b, b)
        return carry

    lax.fori_loop(0, NGROUP, group, 0)
    for k in range(NREM):
        step(NGROUP * NBUF + k, k)
    for k in range(NBUF - 1):
        wait_out((NCHUNK - (NBUF - 1) + k) % NBUF)


def kernel(data, token_table, pos_table):
    data3 = data.reshape(BATCH, 2, HALF).astype(jnp.int32)
    mesh = plsc.VectorSubcoreMesh(core_axis_name="c", subcore_axis_name="s")
    run = functools.partial(
        pl.kernel,
        out_type=jax.ShapeDtypeStruct((BATCH, SEQ_LEN, D_MODEL), jnp.float32),
        mesh=mesh,
        scratch_types=[
            pltpu.VMEM((NCHUNK, 2, HALF), jnp.int32),
            pltpu.VMEM((SEQ_LEN, D_MODEL), jnp.float32),
            pltpu.VMEM((SEQ_LEN, D_MODEL), jnp.float32),
            pltpu.VMEM((SEQ_LEN, D_MODEL), jnp.float32),
            pltpu.VMEM((SEQ_LEN, D_MODEL), jnp.float32),
            pltpu.SemaphoreType.DMA,
            pltpu.SemaphoreType.DMA,
            pltpu.SemaphoreType.DMA,
            pltpu.SemaphoreType.DMA,
            pltpu.SemaphoreType.DMA,
            pltpu.SemaphoreType.DMA,
        ],
    )(_sc_body)
    return run(data3, token_table, pos_table)


# speedup vs baseline: 1.9264x; 1.0157x over previous
import functools

import jax
import jax.numpy as jnp
from jax import lax
from jax.experimental import pallas as pl
from jax.experimental.pallas import tpu as pltpu
from jax.experimental.pallas import tpu_sc as plsc

VOCAB_DIM = 100000
SEQ_LEN = 200
D_MODEL = 128
BATCH = 1024

NC = 2
NS = 16
NW = NC * NS
NCHUNK = BATCH // NW
HALF = SEQ_LEN // 2
NBUF = 3
NGROUP = NCHUNK // NBUF
NREM = NCHUNK - NGROUP * NBUF


def _sc_body(data_hbm, tok_hbm, pos_hbm, out_hbm,
             idx_all, rows0, rows1, rows2, pos_v, g0, g1, g2, o0, o1, o2):
    wid = lax.axis_index("s") * NC + lax.axis_index("c")
    base = wid * NCHUNK
    rows_v = (rows0, rows1, rows2)
    gsem = (g0, g1, g2)
    osem = (o0, o1, o2)

    pltpu.sync_copy(data_hbm.at[pl.ds(base, NCHUNK)], idx_all)
    pltpu.sync_copy(pos_hbm, pos_v)

    def issue_gather(c, b):
        pltpu.async_copy(tok_hbm.at[idx_all.at[c, 0]],
                         rows_v[b].at[pl.ds(0, HALF)], gsem[b])
        pltpu.async_copy(tok_hbm.at[idx_all.at[c, 1]],
                         rows_v[b].at[pl.ds(HALF, HALF)], gsem[b])

    def wait_gather(c, b):
        pltpu.make_async_copy(tok_hbm.at[idx_all.at[c, 0]],
                              rows_v[b].at[pl.ds(0, HALF)], gsem[b]).wait()
        pltpu.make_async_copy(tok_hbm.at[idx_all.at[c, 1]],
                              rows_v[b].at[pl.ds(HALF, HALF)], gsem[b]).wait()

    def wait_out(b):
        pltpu.make_async_copy(rows_v[b], out_hbm.at[base], osem[b]).wait()

    def step(c, b):
        bn = (b + 1) % NBUF

        if isinstance(c, int):
            if c >= NBUF - 1:
                wait_out(bn)
            if c + 1 < NCHUNK:
                issue_gather(c + 1, bn)
        else:
            @pl.when(c >= NBUF - 1)
            def _():
                wait_out(bn)

            @pl.when(c + 1 < NCHUNK)
            def _():
                issue_gather(c + 1, bn)

        wait_gather(c, b)

        @plsc.parallel_loop(0, SEQ_LEN, step=1, unroll=5)
        def addrow(i):
            for j in range(D_MODEL // 16):
                sl = pl.ds(j * 16, 16)
                rows_v[b][i, sl] = rows_v[b][i, sl] + pos_v[i, sl]

        pltpu.async_copy(rows_v[b], out_hbm.at[base + c], osem[b])

    issue_gather(0, 0)

    def group(g, carry):
        for b in range(NBUF):
            step(g * NBUF + b, b)
        return carry

    lax.fori_loop(0, NGROUP, group, 0)
    for k in range(NREM):
        step(NGROUP * NBUF + k, k)
    for k in range(NBUF - 1):
        wait_out((NCHUNK - (NBUF - 1) + k) % NBUF)


def kernel(data, token_table, pos_table):
    data3 = data.reshape(BATCH, 2, HALF).astype(jnp.int32)
    mesh = plsc.VectorSubcoreMesh(core_axis_name="c", subcore_axis_name="s")
    run = functools.partial(
        pl.kernel,
        out_type=jax.ShapeDtypeStruct((BATCH, SEQ_LEN, D_MODEL), jnp.float32),
        mesh=mesh,
        scratch_types=[
            pltpu.VMEM((NCHUNK, 2, HALF), jnp.int32),
            pltpu.VMEM((SEQ_LEN, D_MODEL), jnp.float32),
            pltpu.VMEM((SEQ_LEN, D_MODEL), jnp.float32),
            pltpu.VMEM((SEQ_LEN, D_MODEL), jnp.float32),
            pltpu.VMEM((SEQ_LEN, D_MODEL), jnp.float32),
            pltpu.SemaphoreType.DMA,
            pltpu.SemaphoreType.DMA,
            pltpu.SemaphoreType.DMA,
            pltpu.SemaphoreType.DMA,
            pltpu.SemaphoreType.DMA,
            pltpu.SemaphoreType.DMA,
        ],
    )(_sc_body)
    return run(data3, token_table, pos_table)


# speedup vs baseline: 1.9562x; 1.0155x over previous
import functools

import jax
import jax.numpy as jnp
from jax import lax
from jax.experimental import pallas as pl
from jax.experimental.pallas import tpu as pltpu
from jax.experimental.pallas import tpu_sc as plsc

VOCAB_DIM = 100000
SEQ_LEN = 200
D_MODEL = 128
BATCH = 1024

NC = 2
NS = 16
NW = NC * NS
NCHUNK = BATCH // NW
HALF = SEQ_LEN // 2
NBUF = 3
NGROUP = NCHUNK // NBUF
NREM = NCHUNK - NGROUP * NBUF


def _sc_body(data_hbm, tok_hbm, pos_hbm, out_hbm,
             idx_all, rows0, rows1, rows2, pos_v, g0, g1, g2, o0, o1, o2):
    wid = lax.axis_index("s") * NC + lax.axis_index("c")
    base = wid * NCHUNK
    rows_v = (rows0, rows1, rows2)
    gsem = (g0, g1, g2)
    osem = (o0, o1, o2)

    icp = pltpu.async_copy(data_hbm.at[pl.ds(base, NCHUNK)], idx_all, g0)
    pcp = pltpu.async_copy(pos_hbm, pos_v, o0)
    icp.wait()

    def issue_gather(c, b):
        pltpu.async_copy(tok_hbm.at[idx_all.at[c, 0]],
                         rows_v[b].at[pl.ds(0, HALF)], gsem[b])
        pltpu.async_copy(tok_hbm.at[idx_all.at[c, 1]],
                         rows_v[b].at[pl.ds(HALF, HALF)], gsem[b])

    def wait_gather(c, b):
        pltpu.make_async_copy(tok_hbm.at[idx_all.at[c, 0]],
                              rows_v[b].at[pl.ds(0, HALF)], gsem[b]).wait()
        pltpu.make_async_copy(tok_hbm.at[idx_all.at[c, 1]],
                              rows_v[b].at[pl.ds(HALF, HALF)], gsem[b]).wait()

    def wait_out(b):
        pltpu.make_async_copy(rows_v[b], out_hbm.at[base], osem[b]).wait()

    def step(c, b):
        bn = (b + 1) % NBUF

        if isinstance(c, int):
            if c >= NBUF - 1:
                wait_out(bn)
            if c + 1 < NCHUNK:
                issue_gather(c + 1, bn)
        else:
            @pl.when(c >= NBUF - 1)
            def _():
                wait_out(bn)

            @pl.when(c + 1 < NCHUNK)
            def _():
                issue_gather(c + 1, bn)

        wait_gather(c, b)

        @plsc.parallel_loop(0, SEQ_LEN, step=1, unroll=5)
        def addrow(i):
            for j in range(D_MODEL // 16):
                sl = pl.ds(j * 16, 16)
                rows_v[b][i, sl] = rows_v[b][i, sl] + pos_v[i, sl]

        pltpu.async_copy(rows_v[b], out_hbm.at[base + c], osem[b])

    issue_gather(0, 0)
    pcp.wait()

    def group(g, carry):
        for b in range(NBUF):
            step(g * NBUF + b, b)
        return carry

    lax.fori_loop(0, NGROUP, group, 0)
    for k in range(NREM):
        step(NGROUP * NBUF + k, k)
    for k in range(NBUF - 1):
        wait_out((NCHUNK - (NBUF - 1) + k) % NBUF)


def kernel(data, token_table, pos_table):
    data3 = data.reshape(BATCH, 2, HALF).astype(jnp.int32)
    mesh = plsc.VectorSubcoreMesh(core_axis_name="c", subcore_axis_name="s")
    run = functools.partial(
        pl.kernel,
        out_type=jax.ShapeDtypeStruct((BATCH, SEQ_LEN, D_MODEL), jnp.float32),
        mesh=mesh,
        scratch_types=[
            pltpu.VMEM((NCHUNK, 2, HALF), jnp.int32),
            pltpu.VMEM((SEQ_LEN, D_MODEL), jnp.float32),
            pltpu.VMEM((SEQ_LEN, D_MODEL), jnp.float32),
            pltpu.VMEM((SEQ_LEN, D_MODEL), jnp.float32),
            pltpu.VMEM((SEQ_LEN, D_MODEL), jnp.float32),
            pltpu.SemaphoreType.DMA,
            pltpu.SemaphoreType.DMA,
            pltpu.SemaphoreType.DMA,
            pltpu.SemaphoreType.DMA,
            pltpu.SemaphoreType.DMA,
            pltpu.SemaphoreType.DMA,
        ],
    )(_sc_body)
    return run(data3, token_table, pos_table)
